# baseline (speedup 1.0000x reference)
import jax
import jax.numpy as jnp
from jax import lax
from jax.experimental import pallas as pl
from jax.experimental.pallas import tpu as pltpu
from jax.experimental.pallas import tpu_sc as plsc

NC = 2
NS = 16
NW = NC * NS
PB = 128


def _deg_kernel(npad, base, extra, nfull):
    mesh = plsc.VectorSubcoreMesh(core_axis_name="c", subcore_axis_name="s")
    rpt = npad // NS

    base2 = base // 2
    tail = base - 2 * base2

    def body(cols_hbm, deg_out, idx_v, ones_v, zero_v, deg_sh, sem):
        c = lax.axis_index("c")
        s = lax.axis_index("s")
        wid = c * NS + s
        base_blk = wid * base
        for i in range(PB // 16):
            ones_v[pl.ds(i * 16, 16)] = jnp.ones((16,), jnp.float32)
        for i in range(rpt // 16):
            zero_v[pl.ds(i * 16, 16)] = jnp.zeros((16,), jnp.float32)
        pltpu.sync_copy(zero_v, deg_sh.at[pl.ds(s * rpt, rpt)])
        plsc.subcore_barrier()
        for b in range(2):
            pltpu.async_copy(cols_hbm.at[base_blk + b], idx_v.at[b], sem)

        def pair(k, carry):
            for b in range(2):
                j = 2 * k + b
                pltpu.make_async_copy(cols_hbm.at[base_blk + b],
                                      idx_v.at[b], sem).wait()
                pltpu.sync_copy(ones_v, deg_sh.at[idx_v.at[b]], add=True)

                @pl.when(k < base2 - 1)
                def _():
                    pltpu.async_copy(cols_hbm.at[base_blk + j + 2],
                                     idx_v.at[b], sem)
            return carry

        lax.fori_loop(0, base2, pair, 0)

        def do_block(blkidx):
            pltpu.async_copy(cols_hbm.at[blkidx], idx_v.at[0], sem).wait()
            pltpu.sync_copy(ones_v, deg_sh.at[idx_v.at[0]], add=True)

        if tail:
            do_block(base_blk + 2 * base2)
        if extra:
            @pl.when(wid < extra)
            def _():
                do_block(NW * base + wid)
        plsc.subcore_barrier()
        pltpu.sync_copy(deg_sh.at[pl.ds(s * rpt, rpt)],
                        deg_out.at[pl.ds(c * npad + s * rpt, rpt)])

    return pl.kernel(
        body,
        out_type=jax.ShapeDtypeStruct((NC * npad,), jnp.float32),
        mesh=mesh,
        scratch_types=[
            pltpu.VMEM((2, PB), jnp.int32),
            pltpu.VMEM((PB,), jnp.float32),
            pltpu.VMEM((rpt,), jnp.float32),
            pltpu.VMEM_SHARED((npad,), jnp.float32),
            pltpu.SemaphoreType.DMA,
        ],
    )


def _scatter_kernel(npad, base, extra, nfull, d):
    mesh = plsc.VectorSubcoreMesh(core_axis_name="c", subcore_axis_name="s")
    rpt = npad // NS
    base2 = base // 2
    tail = base - 2 * base2

    def body(rows_hbm, cols_hbm, y_hbm, out_hbm,
             idxr_v, idxc_v, msg_v, acc_sh, sem):
        c = lax.axis_index("c")
        s = lax.axis_index("s")
        wid = c * NS + s
        base_blk = wid * base
        pltpu.sync_copy(y_hbm.at[pl.ds(s * rpt, rpt)],
                        acc_sh.at[pl.ds(s * rpt, rpt)])
        plsc.subcore_barrier()
        for b in range(2):
            pltpu.sync_copy(rows_hbm.at[base_blk + b], idxr_v.at[b])
            pltpu.sync_copy(cols_hbm.at[base_blk + b], idxc_v.at[b])
            pltpu.async_copy(y_hbm.at[idxr_v.at[b]], msg_v.at[b], sem)

        def pair(k, carry):
            for b in range(2):
                j = 2 * k + b
                pltpu.make_async_copy(y_hbm.at[idxr_v.at[b]], msg_v.at[b],
                                      sem).wait()
                pltpu.sync_copy(msg_v.at[b], acc_sh.at[idxc_v.at[b]],
                                add=True)

                @pl.when(k < base2 - 1)
                def _():
                    pltpu.sync_copy(rows_hbm.at[base_blk + j + 2],
                                    idxr_v.at[b])
                    pltpu.sync_copy(cols_hbm.at[base_blk + j + 2],
                                    idxc_v.at[b])
                    pltpu.async_copy(y_hbm.at[idxr_v.at[b]], msg_v.at[b], sem)
            return carry

        lax.fori_loop(0, base2, pair, 0)

        def do_block(blkidx):
            pltpu.sync_copy(rows_hbm.at[blkidx], idxr_v.at[0])
            pltpu.sync_copy(cols_hbm.at[blkidx], idxc_v.at[0])
            pltpu.async_copy(y_hbm.at[idxr_v.at[0]], msg_v.at[0], sem).wait()
            pltpu.sync_copy(msg_v.at[0], acc_sh.at[idxc_v.at[0]], add=True)

        if tail:
            do_block(base_blk + 2 * base2)
        if extra:
            @pl.when(wid < extra)
            def _():
                do_block(NW * base + wid)
        plsc.subcore_barrier()
        pltpu.sync_copy(acc_sh.at[pl.ds(s * rpt, rpt)],
                        out_hbm.at[pl.ds(c * npad + s * rpt, rpt)])

    return pl.kernel(
        body,
        out_type=jax.ShapeDtypeStruct((NC * npad, d), jnp.float32),
        mesh=mesh,
        scratch_types=[
            pltpu.VMEM((2, PB), jnp.int32),
            pltpu.VMEM((2, PB), jnp.int32),
            pltpu.VMEM((2, PB, d), jnp.float32),
            pltpu.VMEM_SHARED((npad, d), jnp.float32),
            pltpu.SemaphoreType.DMA,
        ],
    )


def _matmul_kernel(npad, d_in, d_out, br):
    def body(x_ref, wt_ref, xw_ref):
        xw_ref[...] = jnp.dot(x_ref[...], wt_ref[...],
                              preferred_element_type=jnp.float32)

    return pl.pallas_call(
        body,
        grid=(npad // br,),
        in_specs=[
            pl.BlockSpec((br, d_in), lambda i: (i, 0)),
            pl.BlockSpec((d_in, d_out), lambda i: (0, 0)),
        ],
        out_specs=pl.BlockSpec((br, d_out), lambda i: (i, 0)),
        out_shape=jax.ShapeDtypeStruct((npad, d_out), jnp.float32),
    )


def _scale_kernel(npad, d, br):
    nblk = npad // br

    def body(xw_ref, d0_ref, d1_ref, y_ref):
        deg = d0_ref[...] + d1_ref[...] + 1.0
        dinv = lax.rsqrt(deg)
        y_ref[...] = xw_ref[...] * dinv[:, None]

    return pl.pallas_call(
        body,
        grid=(nblk,),
        in_specs=[
            pl.BlockSpec((br, d), lambda i: (i, 0)),
            pl.BlockSpec((br,), lambda i: (i,)),
            pl.BlockSpec((br,), lambda i: (nblk + i,)),
        ],
        out_specs=pl.BlockSpec((br, d), lambda i: (i, 0)),
        out_shape=jax.ShapeDtypeStruct((npad, d), jnp.float32),
    )


def _final_kernel(n, npad, d, br):
    nblk = npad // br

    def body(acc0_ref, acc1_ref, y_ref, d0_ref, d1_ref, o_ref):
        deg = d0_ref[...] + d1_ref[...] + 1.0
        dinv = lax.rsqrt(deg)
        o_ref[...] = (acc0_ref[...] + acc1_ref[...] - y_ref[...]) * dinv[:, None]

    return pl.pallas_call(
        body,
        grid=(nblk,),
        in_specs=[
            pl.BlockSpec((br, d), lambda i: (i, 0)),
            pl.BlockSpec((br, d), lambda i: (nblk + i, 0)),
            pl.BlockSpec((br, d), lambda i: (i, 0)),
            pl.BlockSpec((br,), lambda i: (i,)),
            pl.BlockSpec((br,), lambda i: (nblk + i,)),
        ],
        out_specs=pl.BlockSpec((br, d), lambda i: (i, 0)),
        out_shape=jax.ShapeDtypeStruct((n, d), jnp.float32),
    )


def kernel(x, edge_index, num_nodes, W):
    n, d_in = x.shape
    d_out = W.shape[0]
    e = edge_index.shape[1]
    del num_nodes

    npad = -(-n // (NS * 16)) * (NS * 16)
    if npad == n:
        npad += NS * 16

    row = edge_index[0]
    col = edge_index[1]
    rem = e % PB
    if rem:
        padn = PB - rem
        pad_iota = jnp.arange(padn, dtype=edge_index.dtype)
        row = jnp.concatenate([row, pad_iota % n])
        col = jnp.concatenate([col, n + pad_iota % (npad - n)])
    nfull = (e + (PB - rem if rem else 0)) // PB
    base = nfull // NW
    extra = nfull % NW
    rows2 = row.reshape(nfull, PB)
    cols2 = col.reshape(nfull, PB)
    wt = W.T

    degp = _deg_kernel(npad, base, extra, nfull)(cols2)

    xw = _matmul_kernel(npad, d_in, d_out, 512)(x, wt)
    y = _scale_kernel(npad, d_out, 512)(xw, degp, degp)

    accp = _scatter_kernel(npad, base, extra, nfull, d_out)(rows2, cols2, y)

    out = _final_kernel(n, npad, d_out, 512)(accp, accp, y, degp, degp)
    return out


# speedup vs baseline: 1.0951x; 1.0951x over previous
import jax
import jax.numpy as jnp
from jax import lax
from jax.experimental import pallas as pl
from jax.experimental.pallas import tpu as pltpu
from jax.experimental.pallas import tpu_sc as plsc

NC = 2
NS = 16
NW = NC * NS
PB = 128


def _deg_kernel(npad, base, extra, nfull):
    mesh = plsc.VectorSubcoreMesh(core_axis_name="c", subcore_axis_name="s")
    rpt = npad // NS

    nbmax = base + (1 if extra else 0)

    def body(ei_hbm, deg_out, idx_v, ones_v, zero_v, deg_sh, sem):
        c = lax.axis_index("c")
        s = lax.axis_index("s")
        wid = c * NS + s
        base_blk = wid * base
        nbt = base + jnp.where(wid < extra, 1, 0) if extra else base
        for i in range(PB // 16):
            ones_v[pl.ds(i * 16, 16)] = jnp.ones((16,), jnp.float32)
        for i in range(rpt // 16):
            zero_v[pl.ds(i * 16, 16)] = jnp.zeros((16,), jnp.float32)
        pltpu.sync_copy(zero_v, deg_sh.at[pl.ds(s * rpt, rpt)])
        def fire(j, carry):
            pltpu.async_copy(ei_hbm.at[1, base_blk + j], idx_v.at[j], sem)
            return carry

        lax.fori_loop(0, base, fire, 0)
        if extra:
            @pl.when(wid < extra)
            def _():
                pltpu.async_copy(ei_hbm.at[1, NW * base + wid],
                                 idx_v.at[base], sem)

        def drain(j, carry):
            pltpu.make_async_copy(ei_hbm.at[1, 0], idx_v.at[0], sem).wait()
            return carry

        lax.fori_loop(0, nbt, drain, 0)
        plsc.subcore_barrier()

        def blk(j, carry):
            pltpu.sync_copy(ones_v, deg_sh.at[idx_v.at[j]], add=True)
            return carry

        lax.fori_loop(0, nbt, blk, 0)
        plsc.subcore_barrier()
        pltpu.sync_copy(deg_sh.at[pl.ds(s * rpt, rpt)],
                        deg_out.at[pl.ds(c * npad + s * rpt, rpt)])

    return pl.kernel(
        body,
        out_type=jax.ShapeDtypeStruct((NC * npad,), jnp.float32),
        mesh=mesh,
        scratch_types=[
            pltpu.VMEM((nbmax, PB), jnp.int32),
            pltpu.VMEM((PB,), jnp.float32),
            pltpu.VMEM((rpt,), jnp.float32),
            pltpu.VMEM_SHARED((npad,), jnp.float32),
            pltpu.SemaphoreType.DMA,
        ],
    )


def _scatter_kernel(npad, base, extra, nfull, d):
    mesh = plsc.VectorSubcoreMesh(core_axis_name="c", subcore_axis_name="s")
    rpt = npad // NS
    base2 = base // 2
    tail = base - 2 * base2

    def body(ei_hbm, y_hbm, out_hbm,
             idxr_v, idxc_v, msg_v, acc_sh, sem):
        c = lax.axis_index("c")
        s = lax.axis_index("s")
        wid = c * NS + s
        base_blk = wid * base
        pltpu.sync_copy(y_hbm.at[pl.ds(s * rpt, rpt)],
                        acc_sh.at[pl.ds(s * rpt, rpt)])
        plsc.subcore_barrier()
        for b in range(2):
            pltpu.sync_copy(ei_hbm.at[0, base_blk + b], idxr_v.at[b])
            pltpu.sync_copy(ei_hbm.at[1, base_blk + b], idxc_v.at[b])
            pltpu.async_copy(y_hbm.at[idxr_v.at[b]], msg_v.at[b], sem)

        def pair(k, carry):
            for b in range(2):
                j = 2 * k + b
                pltpu.make_async_copy(y_hbm.at[idxr_v.at[b]], msg_v.at[b],
                                      sem).wait()
                pltpu.sync_copy(msg_v.at[b], acc_sh.at[idxc_v.at[b]],
                                add=True)

                @pl.when(k < base2 - 1)
                def _():
                    pltpu.sync_copy(ei_hbm.at[0, base_blk + j + 2],
                                    idxr_v.at[b])
                    pltpu.sync_copy(ei_hbm.at[1, base_blk + j + 2],
                                    idxc_v.at[b])
                    pltpu.async_copy(y_hbm.at[idxr_v.at[b]], msg_v.at[b], sem)
            return carry

        lax.fori_loop(0, base2, pair, 0)

        def do_block(blkidx):
            pltpu.sync_copy(ei_hbm.at[0, blkidx], idxr_v.at[0])
            pltpu.sync_copy(ei_hbm.at[1, blkidx], idxc_v.at[0])
            pltpu.async_copy(y_hbm.at[idxr_v.at[0]], msg_v.at[0], sem).wait()
            pltpu.sync_copy(msg_v.at[0], acc_sh.at[idxc_v.at[0]], add=True)

        if tail:
            do_block(base_blk + 2 * base2)
        if extra:
            @pl.when(wid < extra)
            def _():
                do_block(NW * base + wid)
        plsc.subcore_barrier()
        pltpu.sync_copy(acc_sh.at[pl.ds(s * rpt, rpt)],
                        out_hbm.at[pl.ds(c * npad + s * rpt, rpt)])

    return pl.kernel(
        body,
        out_type=jax.ShapeDtypeStruct((NC * npad, d), jnp.float32),
        mesh=mesh,
        scratch_types=[
            pltpu.VMEM((2, PB), jnp.int32),
            pltpu.VMEM((2, PB), jnp.int32),
            pltpu.VMEM((2, PB, d), jnp.float32),
            pltpu.VMEM_SHARED((npad, d), jnp.float32),
            pltpu.SemaphoreType.DMA,
        ],
    )


def _matmul_kernel(npad, d_in, d_out, br):
    def body(x_ref, wt_ref, xw_ref):
        xw_ref[...] = jnp.dot(x_ref[...], wt_ref[...],
                              preferred_element_type=jnp.float32)

    return pl.pallas_call(
        body,
        grid=(npad // br,),
        in_specs=[
            pl.BlockSpec((br, d_in), lambda i: (i, 0)),
            pl.BlockSpec((d_in, d_out), lambda i: (0, 0)),
        ],
        out_specs=pl.BlockSpec((br, d_out), lambda i: (i, 0)),
        out_shape=jax.ShapeDtypeStruct((npad, d_out), jnp.float32),
    )


def _scale_kernel(npad, d, br):
    nblk = npad // br

    def body(xw_ref, d0_ref, d1_ref, y_ref):
        deg = d0_ref[...] + d1_ref[...] + 1.0
        dinv = lax.rsqrt(deg)
        y_ref[...] = xw_ref[...] * dinv[:, None]

    return pl.pallas_call(
        body,
        grid=(nblk,),
        in_specs=[
            pl.BlockSpec((br, d), lambda i: (i, 0)),
            pl.BlockSpec((br,), lambda i: (i,)),
            pl.BlockSpec((br,), lambda i: (nblk + i,)),
        ],
        out_specs=pl.BlockSpec((br, d), lambda i: (i, 0)),
        out_shape=jax.ShapeDtypeStruct((npad, d), jnp.float32),
    )


def _final_kernel(n, npad, d, br):
    nblk = npad // br

    def body(acc0_ref, acc1_ref, y_ref, d0_ref, d1_ref, o_ref):
        deg = d0_ref[...] + d1_ref[...] + 1.0
        dinv = lax.rsqrt(deg)
        o_ref[...] = (acc0_ref[...] + acc1_ref[...] - y_ref[...]) * dinv[:, None]

    return pl.pallas_call(
        body,
        grid=(nblk,),
        in_specs=[
            pl.BlockSpec((br, d), lambda i: (i, 0)),
            pl.BlockSpec((br, d), lambda i: (nblk + i, 0)),
            pl.BlockSpec((br, d), lambda i: (i, 0)),
            pl.BlockSpec((br,), lambda i: (i,)),
            pl.BlockSpec((br,), lambda i: (nblk + i,)),
        ],
        out_specs=pl.BlockSpec((br, d), lambda i: (i, 0)),
        out_shape=jax.ShapeDtypeStruct((n, d), jnp.float32),
    )


def kernel(x, edge_index, num_nodes, W):
    n, d_in = x.shape
    d_out = W.shape[0]
    e = edge_index.shape[1]
    del num_nodes

    npad = -(-n // (NS * 16)) * (NS * 16)
    if npad == n:
        npad += NS * 16

    rem = e % PB
    if rem:
        padn = PB - rem
        pad_iota = jnp.arange(padn, dtype=edge_index.dtype)
        pad_blk = jnp.stack([pad_iota % n, n + pad_iota % (npad - n)])
        edge_index = jnp.concatenate([edge_index, pad_blk], axis=1)
    nfull = (e + (PB - rem if rem else 0)) // PB
    base = nfull // NW
    extra = nfull % NW
    ei3 = edge_index.reshape(2, nfull, PB)
    wt = W.T

    degp = _deg_kernel(npad, base, extra, nfull)(ei3)

    xw = _matmul_kernel(npad, d_in, d_out, 512)(x, wt)
    y = _scale_kernel(npad, d_out, 512)(xw, degp, degp)

    accp = _scatter_kernel(npad, base, extra, nfull, d_out)(ei3, y)

    out = _final_kernel(n, npad, d_out, 512)(accp, accp, y, degp, degp)
    return out


# speedup vs baseline: 1.1288x; 1.0308x over previous
import jax
import jax.numpy as jnp
from jax import lax
from jax.experimental import pallas as pl
from jax.experimental.pallas import tpu as pltpu
from jax.experimental.pallas import tpu_sc as plsc

NC = 2
NS = 16
NW = NC * NS
PB = 128


def _ds(off):
    return pl.ds(pl.multiple_of(off, PB), PB)


def _deg_kernel(npad, base, extra):
    mesh = plsc.VectorSubcoreMesh(core_axis_name="c", subcore_axis_name="s")
    rpt = npad // NS
    nbmax = base + (1 if extra else 0)

    def body(ei_hbm, deg_out, idx_v, ones_v, zero_v, deg_sh, sem):
        c = lax.axis_index("c")
        s = lax.axis_index("s")
        wid = c * NS + s
        base_blk = wid * base
        nbt = base + jnp.where(wid < extra, 1, 0) if extra else base
        for i in range(PB // 16):
            ones_v[pl.ds(i * 16, 16)] = jnp.ones((16,), jnp.float32)
        for i in range(rpt // 16):
            zero_v[pl.ds(i * 16, 16)] = jnp.zeros((16,), jnp.float32)
        pltpu.sync_copy(zero_v, deg_sh.at[pl.ds(s * rpt, rpt)])
        def fire(j, carry):
            pltpu.async_copy(ei_hbm.at[1, _ds((base_blk + j) * PB)],
                             idx_v.at[j], sem)
            return carry

        lax.fori_loop(0, base, fire, 0)
        if extra:
            @pl.when(wid < extra)
            def _():
                pltpu.async_copy(ei_hbm.at[1, _ds((NW * base + wid) * PB)],
                                 idx_v.at[base], sem)

        def drain(j, carry):
            pltpu.make_async_copy(ei_hbm.at[1, _ds(0)], idx_v.at[0],
                                  sem).wait()
            return carry

        lax.fori_loop(0, nbt, drain, 0)
        plsc.subcore_barrier()

        def blk(j, carry):
            pltpu.sync_copy(ones_v, deg_sh.at[idx_v.at[j]], add=True)
            return carry

        lax.fori_loop(0, nbt, blk, 0)
        plsc.subcore_barrier()
        pltpu.sync_copy(deg_sh.at[pl.ds(s * rpt, rpt)],
                        deg_out.at[pl.ds(c * npad + s * rpt, rpt)])

    return pl.kernel(
        body,
        out_type=jax.ShapeDtypeStruct((NC * npad,), jnp.float32),
        mesh=mesh,
        scratch_types=[
            pltpu.VMEM((nbmax, PB), jnp.int32),
            pltpu.VMEM((PB,), jnp.float32),
            pltpu.VMEM((rpt,), jnp.float32),
            pltpu.VMEM_SHARED((npad,), jnp.float32),
            pltpu.SemaphoreType.DMA,
        ],
    )


def _scatter_kernel(npad, base, extra, d):
    mesh = plsc.VectorSubcoreMesh(core_axis_name="c", subcore_axis_name="s")
    rpt = npad // NS
    base2 = base // 2
    tail = base - 2 * base2

    def body(ei_hbm, y_hbm, out_hbm,
             idxr0_v, idxr1_v, idxc0_v, idxc1_v, msg_v, acc_sh, sem):
        idxr = (idxr0_v, idxr1_v)
        idxc = (idxc0_v, idxc1_v)
        c = lax.axis_index("c")
        s = lax.axis_index("s")
        wid = c * NS + s
        base_blk = wid * base
        @pl.when(c == 0)
        def _():
            pltpu.sync_copy(y_hbm.at[pl.ds(s * rpt, rpt)],
                            acc_sh.at[pl.ds(s * rpt, rpt)])

        @pl.when(c != 0)
        def _():
            def zrow(r, carry):
                for i in range(d // 16):
                    msg_v[0, r, pl.ds(i * 16, 16)] = jnp.zeros(
                        (16,), jnp.float32)
                return carry

            lax.fori_loop(0, PB, zrow, 0)
            for r in range(rpt // PB):
                pltpu.sync_copy(
                    msg_v.at[0], acc_sh.at[pl.ds(s * rpt + r * PB, PB)])

        plsc.subcore_barrier()
        for b in range(2):
            off = pl.multiple_of((base_blk + b) * PB, PB)
            pltpu.sync_copy(ei_hbm.at[0, pl.ds(off, PB)], idxr[b])
            pltpu.sync_copy(ei_hbm.at[1, pl.ds(off, PB)], idxc[b])
            pltpu.async_copy(y_hbm.at[idxr[b]], msg_v.at[b], sem)

        def pair(k, carry):
            for b in range(2):
                j = 2 * k + b
                pltpu.make_async_copy(y_hbm.at[idxr[b]], msg_v.at[b],
                                      sem).wait()
                pltpu.sync_copy(msg_v.at[b], acc_sh.at[idxc[b]],
                                add=True)

                @pl.when(k < base2 - 1)
                def _():
                    off2 = pl.multiple_of((base_blk + j + 2) * PB, PB)
                    pltpu.sync_copy(ei_hbm.at[0, pl.ds(off2, PB)], idxr[b])
                    pltpu.sync_copy(ei_hbm.at[1, pl.ds(off2, PB)], idxc[b])
                    pltpu.async_copy(y_hbm.at[idxr[b]], msg_v.at[b], sem)
            return carry

        lax.fori_loop(0, base2, pair, 0)

        def do_block(blkidx):
            pltpu.sync_copy(ei_hbm.at[0, _ds(blkidx * PB)], idxr[0])
            pltpu.sync_copy(ei_hbm.at[1, _ds(blkidx * PB)], idxc[0])
            pltpu.async_copy(y_hbm.at[idxr[0]], msg_v.at[0], sem).wait()
            pltpu.sync_copy(msg_v.at[0], acc_sh.at[idxc[0]], add=True)

        if tail:
            do_block(base_blk + 2 * base2)
        if extra:
            @pl.when(wid < extra)
            def _():
                do_block(NW * base + wid)
        plsc.subcore_barrier()
        pltpu.sync_copy(acc_sh.at[pl.ds(s * rpt, rpt)],
                        out_hbm.at[pl.ds(c * npad + s * rpt, rpt)])

    return pl.kernel(
        body,
        out_type=jax.ShapeDtypeStruct((NC * npad, d), jnp.float32),
        mesh=mesh,
        scratch_types=[
            pltpu.VMEM((PB,), jnp.int32),
            pltpu.VMEM((PB,), jnp.int32),
            pltpu.VMEM((PB,), jnp.int32),
            pltpu.VMEM((PB,), jnp.int32),
            pltpu.VMEM((2, PB, d), jnp.float32),
            pltpu.VMEM_SHARED((npad, d), jnp.float32),
            pltpu.SemaphoreType.DMA,
        ],
    )


def _matmul_kernel(npad, d_in, d_out, br):
    def body(x_ref, wt_ref, xw_ref):
        xw_ref[...] = jnp.dot(x_ref[...], wt_ref[...],
                              preferred_element_type=jnp.float32)

    return pl.pallas_call(
        body,
        grid=(npad // br,),
        in_specs=[
            pl.BlockSpec((br, d_in), lambda i: (i, 0)),
            pl.BlockSpec((d_in, d_out), lambda i: (0, 0)),
        ],
        out_specs=pl.BlockSpec((br, d_out), lambda i: (i, 0)),
        out_shape=jax.ShapeDtypeStruct((npad, d_out), jnp.float32),
    )


def _scale_kernel(npad, d, br):
    nblk = npad // br

    def body(xw_ref, d0_ref, d1_ref, y_ref):
        deg = d0_ref[...] + d1_ref[...] + 1.0
        dinv = lax.rsqrt(deg)
        y_ref[...] = xw_ref[...] * dinv[:, None]

    return pl.pallas_call(
        body,
        grid=(nblk,),
        in_specs=[
            pl.BlockSpec((br, d), lambda i: (i, 0)),
            pl.BlockSpec((br,), lambda i: (i,)),
            pl.BlockSpec((br,), lambda i: (nblk + i,)),
        ],
        out_specs=pl.BlockSpec((br, d), lambda i: (i, 0)),
        out_shape=jax.ShapeDtypeStruct((npad, d), jnp.float32),
    )


def _final_kernel(n, npad, d, br):
    nblk = npad // br

    def body(acc0_ref, acc1_ref, d0_ref, d1_ref, o_ref):
        deg = d0_ref[...] + d1_ref[...] + 1.0
        dinv = lax.rsqrt(deg)
        o_ref[...] = (acc0_ref[...] + acc1_ref[...]) * dinv[:, None]

    return pl.pallas_call(
        body,
        grid=(nblk,),
        in_specs=[
            pl.BlockSpec((br, d), lambda i: (i, 0)),
            pl.BlockSpec((br, d), lambda i: (nblk + i, 0)),
            pl.BlockSpec((br,), lambda i: (i,)),
            pl.BlockSpec((br,), lambda i: (nblk + i,)),
        ],
        out_specs=pl.BlockSpec((br, d), lambda i: (i, 0)),
        out_shape=jax.ShapeDtypeStruct((n, d), jnp.float32),
    )


def kernel(x, edge_index, num_nodes, W):
    n, d_in = x.shape
    d_out = W.shape[0]
    e = edge_index.shape[1]
    del num_nodes

    npad = -(-n // (NS * 16)) * (NS * 16)
    if npad == n:
        npad += NS * 16

    rem = e % PB
    if rem:
        padn = PB - rem
        pad_iota = jnp.arange(padn, dtype=edge_index.dtype)
        pad_blk = jnp.stack([pad_iota % n, n + pad_iota % (npad - n)])
        edge_index = jnp.concatenate([edge_index, pad_blk], axis=1)
    nfull = (e + (PB - rem if rem else 0)) // PB
    base = nfull // NW
    extra = nfull % NW
    wt = W.T

    degp = _deg_kernel(npad, base, extra)(edge_index)

    xw = _matmul_kernel(npad, d_in, d_out, 512)(x, wt)
    y = _scale_kernel(npad, d_out, 512)(xw, degp, degp)

    accp = _scatter_kernel(npad, base, extra, d_out)(edge_index, y)

    out = _final_kernel(n, npad, d_out, 512)(accp, accp, degp, degp)
    return out


# speedup vs baseline: 1.4138x; 1.2525x over previous
import jax
import jax.numpy as jnp
from jax import lax
from jax.experimental import pallas as pl
from jax.experimental.pallas import tpu as pltpu
from jax.experimental.pallas import tpu_sc as plsc

NC = 2
NS = 16
NW = NC * NS
PB = 128


def _ds(off):
    return pl.ds(pl.multiple_of(off, PB), PB)


def _deg_kernel(npad, base, extra):
    mesh = plsc.VectorSubcoreMesh(core_axis_name="c", subcore_axis_name="s")
    rpt = npad // NS
    nbmax = base + (1 if extra else 0)

    def body(ei_hbm, deg_out, idx_v, ones_v, zero_v, deg_sh, sem):
        c = lax.axis_index("c")
        s = lax.axis_index("s")
        wid = c * NS + s
        base_blk = wid * base
        nbt = base + jnp.where(wid < extra, 1, 0) if extra else base
        for i in range(PB // 16):
            ones_v[pl.ds(i * 16, 16)] = jnp.ones((16,), jnp.float32)
        for i in range(rpt // 16):
            zero_v[pl.ds(i * 16, 16)] = jnp.zeros((16,), jnp.float32)
        pltpu.sync_copy(zero_v, deg_sh.at[pl.ds(s * rpt, rpt)])
        def fire(j, carry):
            pltpu.async_copy(ei_hbm.at[1, _ds((base_blk + j) * PB)],
                             idx_v.at[j], sem)
            return carry

        lax.fori_loop(0, base, fire, 0)
        if extra:
            @pl.when(wid < extra)
            def _():
                pltpu.async_copy(ei_hbm.at[1, _ds((NW * base + wid) * PB)],
                                 idx_v.at[base], sem)

        def drain(j, carry):
            pltpu.make_async_copy(ei_hbm.at[1, _ds(0)], idx_v.at[0],
                                  sem).wait()
            return carry

        lax.fori_loop(0, nbt, drain, 0)
        plsc.subcore_barrier()

        def blk(j, carry):
            pltpu.sync_copy(ones_v, deg_sh.at[idx_v.at[j]], add=True)
            return carry

        lax.fori_loop(0, nbt, blk, 0)
        plsc.subcore_barrier()
        pltpu.sync_copy(deg_sh.at[pl.ds(s * rpt, rpt)],
                        deg_out.at[pl.ds(c * npad + s * rpt, rpt)])

    return pl.kernel(
        body,
        out_type=jax.ShapeDtypeStruct((NC * npad,), jnp.float32),
        mesh=mesh,
        scratch_types=[
            pltpu.VMEM((nbmax, PB), jnp.int32),
            pltpu.VMEM((PB,), jnp.float32),
            pltpu.VMEM((rpt,), jnp.float32),
            pltpu.VMEM_SHARED((npad,), jnp.float32),
            pltpu.SemaphoreType.DMA,
        ],
    )


def _scatter_kernel(npad, base, extra, d):
    mesh = plsc.VectorSubcoreMesh(core_axis_name="c", subcore_axis_name="s")
    rpt = npad // NS
    nbq = base // 4
    tailq = base - 4 * nbq

    def body(ei_hbm, y_hbm, out_hbm,
             idxr0_v, idxr1_v, idxr2_v, idxr3_v,
             idxc0_v, idxc1_v, idxc2_v, idxc3_v, msg_v, acc_sh,
             sem, semi0, semi1, semi2, semi3):
        idxr = (idxr0_v, idxr1_v, idxr2_v, idxr3_v)
        idxc = (idxc0_v, idxc1_v, idxc2_v, idxc3_v)
        semi = (semi0, semi1, semi2, semi3)
        c = lax.axis_index("c")
        s = lax.axis_index("s")
        wid = c * NS + s
        base_blk = wid * base
        @pl.when(c == 0)
        def _():
            pltpu.sync_copy(y_hbm.at[pl.ds(s * rpt, rpt)],
                            acc_sh.at[pl.ds(s * rpt, rpt)])

        @pl.when(c != 0)
        def _():
            def zrow(r, carry):
                for i in range(d // 16):
                    msg_v[0, r, pl.ds(i * 16, 16)] = jnp.zeros(
                        (16,), jnp.float32)
                return carry

            lax.fori_loop(0, PB, zrow, 0)
            for r in range(rpt // PB):
                pltpu.sync_copy(
                    msg_v.at[0], acc_sh.at[pl.ds(s * rpt + r * PB, PB)])

        plsc.subcore_barrier()

        def off(j):
            return pl.multiple_of((base_blk + j) * PB, PB)

        def start_idx(slot, j, sm):
            pltpu.async_copy(ei_hbm.at[0, pl.ds(off(j), PB)], idxr[slot], sm)
            pltpu.async_copy(ei_hbm.at[1, pl.ds(off(j), PB)], idxc[slot], sm)

        def wait_idx(slot, sm):
            pltpu.make_async_copy(ei_hbm.at[0, _ds(0)], idxr[slot], sm).wait()
            pltpu.make_async_copy(ei_hbm.at[1, _ds(0)], idxc[slot], sm).wait()

        for b in range(2):
            pltpu.sync_copy(ei_hbm.at[0, pl.ds(off(b), PB)], idxr[b])
            pltpu.sync_copy(ei_hbm.at[1, pl.ds(off(b), PB)], idxc[b])
            pltpu.async_copy(y_hbm.at[idxr[b]], msg_v.at[b], sem)
        for b in range(2, 4):
            if base > b:
                start_idx(b, b, semi[b])

        def quad(q, carry):
            for b4 in range(4):
                j = 4 * q + b4
                ms = b4 % 2
                pltpu.make_async_copy(y_hbm.at[idxr[b4]], msg_v.at[ms],
                                      sem).wait()
                pltpu.sync_copy(msg_v.at[ms], acc_sh.at[idxc[b4]],
                                add=True)
                i2 = (b4 + 2) % 4

                @pl.when(j + 2 < base)
                def _():
                    wait_idx(i2, semi[i2])
                    pltpu.async_copy(y_hbm.at[idxr[i2]], msg_v.at[ms], sem)

                @pl.when(j + 4 < base)
                def _():
                    start_idx(b4, j + 4, semi[b4])
            return carry

        lax.fori_loop(0, nbq, quad, 0)
        for t in range(tailq):
            pltpu.make_async_copy(y_hbm.at[idxr[t]], msg_v.at[t % 2],
                                  sem).wait()
            pltpu.sync_copy(msg_v.at[t % 2], acc_sh.at[idxc[t]], add=True)

        def do_block(blkidx):
            pltpu.sync_copy(ei_hbm.at[0, _ds(blkidx * PB)], idxr[0])
            pltpu.sync_copy(ei_hbm.at[1, _ds(blkidx * PB)], idxc[0])
            pltpu.async_copy(y_hbm.at[idxr[0]], msg_v.at[0], sem).wait()
            pltpu.sync_copy(msg_v.at[0], acc_sh.at[idxc[0]], add=True)

        if extra:
            @pl.when(wid < extra)
            def _():
                do_block(NW * base + wid)
        plsc.subcore_barrier()
        pltpu.sync_copy(acc_sh.at[pl.ds(s * rpt, rpt)],
                        out_hbm.at[pl.ds(c * npad + s * rpt, rpt)])

    return pl.kernel(
        body,
        out_type=jax.ShapeDtypeStruct((NC * npad, d), jnp.float32),
        mesh=mesh,
        scratch_types=(
            [pltpu.VMEM((PB,), jnp.int32)] * 8
            + [
                pltpu.VMEM((2, PB, d), jnp.float32),
                pltpu.VMEM_SHARED((npad, d), jnp.float32),
            ]
            + [pltpu.SemaphoreType.DMA] * 5
        ),
    )


def _matmul_kernel(npad, d_in, d_out, br):
    def body(x_ref, wt_ref, xw_ref):
        xw_ref[...] = jnp.dot(x_ref[...], wt_ref[...],
                              preferred_element_type=jnp.float32)

    return pl.pallas_call(
        body,
        grid=(npad // br,),
        in_specs=[
            pl.BlockSpec((br, d_in), lambda i: (i, 0)),
            pl.BlockSpec((d_in, d_out), lambda i: (0, 0)),
        ],
        out_specs=pl.BlockSpec((br, d_out), lambda i: (i, 0)),
        out_shape=jax.ShapeDtypeStruct((npad, d_out), jnp.float32),
    )


def _scale_kernel(npad, d, br):
    nblk = npad // br

    def body(xw_ref, d0_ref, d1_ref, y_ref):
        deg = d0_ref[...] + d1_ref[...] + 1.0
        dinv = lax.rsqrt(deg)
        y_ref[...] = xw_ref[...] * dinv[:, None]

    return pl.pallas_call(
        body,
        grid=(nblk,),
        in_specs=[
            pl.BlockSpec((br, d), lambda i: (i, 0)),
            pl.BlockSpec((br,), lambda i: (i,)),
            pl.BlockSpec((br,), lambda i: (nblk + i,)),
        ],
        out_specs=pl.BlockSpec((br, d), lambda i: (i, 0)),
        out_shape=jax.ShapeDtypeStruct((npad, d), jnp.float32),
    )


def _final_kernel(n, npad, d, br):
    nblk = npad // br

    def body(acc0_ref, acc1_ref, d0_ref, d1_ref, o_ref):
        deg = d0_ref[...] + d1_ref[...] + 1.0
        dinv = lax.rsqrt(deg)
        o_ref[...] = (acc0_ref[...] + acc1_ref[...]) * dinv[:, None]

    return pl.pallas_call(
        body,
        grid=(nblk,),
        in_specs=[
            pl.BlockSpec((br, d), lambda i: (i, 0)),
            pl.BlockSpec((br, d), lambda i: (nblk + i, 0)),
            pl.BlockSpec((br,), lambda i: (i,)),
            pl.BlockSpec((br,), lambda i: (nblk + i,)),
        ],
        out_specs=pl.BlockSpec((br, d), lambda i: (i, 0)),
        out_shape=jax.ShapeDtypeStruct((n, d), jnp.float32),
    )


def kernel(x, edge_index, num_nodes, W):
    n, d_in = x.shape
    d_out = W.shape[0]
    e = edge_index.shape[1]
    del num_nodes

    npad = -(-n // (NS * 16)) * (NS * 16)
    if npad == n:
        npad += NS * 16

    rem = e % PB
    if rem:
        padn = PB - rem
        pad_iota = jnp.arange(padn, dtype=edge_index.dtype)
        pad_blk = jnp.stack([pad_iota % n, n + pad_iota % (npad - n)])
        edge_index = jnp.concatenate([edge_index, pad_blk], axis=1)
    nfull = (e + (PB - rem if rem else 0)) // PB
    base = nfull // NW
    extra = nfull % NW
    wt = W.T

    degp = _deg_kernel(npad, base, extra)(edge_index)

    xw = _matmul_kernel(npad, d_in, d_out, 512)(x, wt)
    y = _scale_kernel(npad, d_out, 512)(xw, degp, degp)

    accp = _scatter_kernel(npad, base, extra, d_out)(edge_index, y)

    out = _final_kernel(n, npad, d_out, 512)(accp, accp, degp, degp)
    return out


# speedup vs baseline: 1.4330x; 1.0136x over previous
import jax
import jax.numpy as jnp
from jax import lax
from jax.experimental import pallas as pl
from jax.experimental.pallas import tpu as pltpu
from jax.experimental.pallas import tpu_sc as plsc

NC = 2
NS = 16
NW = NC * NS
PB = 128


def _ds(off):
    return pl.ds(pl.multiple_of(off, PB), PB)


def _deg_kernel(npad, base, extra):
    mesh = plsc.VectorSubcoreMesh(core_axis_name="c", subcore_axis_name="s")
    rpt = npad // NS
    nbmax = base + (1 if extra else 0)

    KD = 4
    sbase = base // KD
    btail = base - sbase * KD

    def body(ei_hbm, deg_out, idx_v, ones_v, zero_v, deg_sh, sem):
        c = lax.axis_index("c")
        s = lax.axis_index("s")
        wid = c * NS + s
        base_blk = wid * base
        take_extra = s * NC + c < extra
        for i in range(KD * PB // 16):
            ones_v[pl.ds(i * 16, 16)] = jnp.ones((16,), jnp.float32)
        for i in range(rpt // 16):
            zero_v[pl.ds(i * 16, 16)] = jnp.zeros((16,), jnp.float32)
        pltpu.sync_copy(zero_v, deg_sh.at[pl.ds(s * rpt, rpt)])
        pltpu.async_copy(
            ei_hbm.at[1, pl.ds(pl.multiple_of(base_blk * PB, PB), base * PB)],
            idx_v.at[pl.ds(0, base * PB)], sem)
        if extra:
            @pl.when(take_extra)
            def _():
                pltpu.async_copy(
                    ei_hbm.at[1, _ds((NW * base + s * NC + c) * PB)],
                    idx_v.at[pl.ds(base * PB, PB)], sem)
            pltpu.make_async_copy(
                ei_hbm.at[1, pl.ds(0, base * PB)],
                idx_v.at[pl.ds(0, base * PB)], sem).wait()
            @pl.when(take_extra)
            def _():
                pltpu.make_async_copy(ei_hbm.at[1, _ds(0)],
                                      idx_v.at[pl.ds(0, PB)], sem).wait()
        else:
            pltpu.make_async_copy(
                ei_hbm.at[1, pl.ds(0, base * PB)],
                idx_v.at[pl.ds(0, base * PB)], sem).wait()
        plsc.subcore_barrier()

        def blk(g, carry):
            pltpu.sync_copy(
                ones_v,
                deg_sh.at[idx_v.at[pl.ds(pl.multiple_of(g * (KD * PB),
                                                        KD * PB), KD * PB)]],
                add=True)
            return carry

        lax.fori_loop(0, sbase, blk, 0)
        for t in range(btail):
            pltpu.sync_copy(
                ones_v.at[pl.ds(0, PB)],
                deg_sh.at[idx_v.at[pl.ds((sbase * KD + t) * PB, PB)]],
                add=True)
        if extra:
            @pl.when(take_extra)
            def _():
                pltpu.sync_copy(
                    ones_v.at[pl.ds(0, PB)],
                    deg_sh.at[idx_v.at[pl.ds(base * PB, PB)]], add=True)
        plsc.subcore_barrier()
        pltpu.sync_copy(deg_sh.at[pl.ds(s * rpt, rpt)],
                        deg_out.at[pl.ds(c * npad + s * rpt, rpt)])

    return pl.kernel(
        body,
        out_type=jax.ShapeDtypeStruct((NC * npad,), jnp.float32),
        mesh=mesh,
        scratch_types=[
            pltpu.VMEM((nbmax * PB,), jnp.int32),
            pltpu.VMEM((KD * PB,), jnp.float32),
            pltpu.VMEM((rpt,), jnp.float32),
            pltpu.VMEM_SHARED((npad,), jnp.float32),
            pltpu.SemaphoreType.DMA,
        ],
    )


def _scatter_kernel(npad, base, extra, d):
    mesh = plsc.VectorSubcoreMesh(core_axis_name="c", subcore_axis_name="s")
    rpt = npad // NS
    nbq = base // 4
    tailq = base - 4 * nbq

    def body(ei_hbm, y_hbm, out_hbm,
             idxr0_v, idxr1_v, idxr2_v, idxr3_v,
             idxc0_v, idxc1_v, idxc2_v, idxc3_v, msg_v, acc_sh,
             sem, semi0, semi1, semi2, semi3):
        idxr = (idxr0_v, idxr1_v, idxr2_v, idxr3_v)
        idxc = (idxc0_v, idxc1_v, idxc2_v, idxc3_v)
        semi = (semi0, semi1, semi2, semi3)
        c = lax.axis_index("c")
        s = lax.axis_index("s")
        wid = c * NS + s
        base_blk = wid * base
        use_y = (c == 0) == (s < NS // 2)

        @pl.when(use_y)
        def _():
            pltpu.sync_copy(y_hbm.at[pl.ds(s * rpt, rpt)],
                            acc_sh.at[pl.ds(s * rpt, rpt)])

        @pl.when(jnp.logical_not(use_y))
        def _():
            def zrow(r, carry):
                for i in range(d // 16):
                    msg_v[0, r, pl.ds(i * 16, 16)] = jnp.zeros(
                        (16,), jnp.float32)
                return carry

            lax.fori_loop(0, PB, zrow, 0)
            for r in range(rpt // PB):
                pltpu.sync_copy(
                    msg_v.at[0], acc_sh.at[pl.ds(s * rpt + r * PB, PB)])

        plsc.subcore_barrier()

        def off(j):
            return pl.multiple_of((base_blk + j) * PB, PB)

        def start_idx(slot, j, sm):
            pltpu.async_copy(ei_hbm.at[0, pl.ds(off(j), PB)], idxr[slot], sm)
            pltpu.async_copy(ei_hbm.at[1, pl.ds(off(j), PB)], idxc[slot], sm)

        def wait_idx(slot, sm):
            pltpu.make_async_copy(ei_hbm.at[0, _ds(0)], idxr[slot], sm).wait()
            pltpu.make_async_copy(ei_hbm.at[1, _ds(0)], idxc[slot], sm).wait()

        for b in range(2):
            pltpu.sync_copy(ei_hbm.at[0, pl.ds(off(b), PB)], idxr[b])
            pltpu.sync_copy(ei_hbm.at[1, pl.ds(off(b), PB)], idxc[b])
            pltpu.async_copy(y_hbm.at[idxr[b]], msg_v.at[b], sem)
        for b in range(2, 4):
            if base > b:
                start_idx(b, b, semi[b])

        def quad(q, carry):
            for b4 in range(4):
                j = 4 * q + b4
                ms = b4 % 2
                pltpu.make_async_copy(y_hbm.at[idxr[b4]], msg_v.at[ms],
                                      sem).wait()
                pltpu.sync_copy(msg_v.at[ms], acc_sh.at[idxc[b4]],
                                add=True)
                i2 = (b4 + 2) % 4

                @pl.when(j + 2 < base)
                def _():
                    wait_idx(i2, semi[i2])
                    pltpu.async_copy(y_hbm.at[idxr[i2]], msg_v.at[ms], sem)

                @pl.when(j + 4 < base)
                def _():
                    start_idx(b4, j + 4, semi[b4])
            return carry

        lax.fori_loop(0, nbq, quad, 0)
        for t in range(tailq):
            pltpu.make_async_copy(y_hbm.at[idxr[t]], msg_v.at[t % 2],
                                  sem).wait()
            pltpu.sync_copy(msg_v.at[t % 2], acc_sh.at[idxc[t]], add=True)

        def do_block(blkidx):
            pltpu.sync_copy(ei_hbm.at[0, _ds(blkidx * PB)], idxr[0])
            pltpu.sync_copy(ei_hbm.at[1, _ds(blkidx * PB)], idxc[0])
            pltpu.async_copy(y_hbm.at[idxr[0]], msg_v.at[0], sem).wait()
            pltpu.sync_copy(msg_v.at[0], acc_sh.at[idxc[0]], add=True)

        if extra:
            @pl.when(s * NC + c < extra)
            def _():
                do_block(NW * base + s * NC + c)
        plsc.subcore_barrier()
        pltpu.sync_copy(acc_sh.at[pl.ds(s * rpt, rpt)],
                        out_hbm.at[pl.ds(c * npad + s * rpt, rpt)])

    return pl.kernel(
        body,
        out_type=jax.ShapeDtypeStruct((NC * npad, d), jnp.float32),
        mesh=mesh,
        scratch_types=(
            [pltpu.VMEM((PB,), jnp.int32)] * 8
            + [
                pltpu.VMEM((2, PB, d), jnp.float32),
                pltpu.VMEM_SHARED((npad, d), jnp.float32),
            ]
            + [pltpu.SemaphoreType.DMA] * 5
        ),
    )


def _matmul_kernel(npad, d_in, d_out, br):
    def body(x_ref, wt_ref, xw_ref):
        xw_ref[...] = jnp.dot(x_ref[...], wt_ref[...],
                              preferred_element_type=jnp.float32)

    return pl.pallas_call(
        body,
        grid=(npad // br,),
        in_specs=[
            pl.BlockSpec((br, d_in), lambda i: (i, 0)),
            pl.BlockSpec((d_in, d_out), lambda i: (0, 0)),
        ],
        out_specs=pl.BlockSpec((br, d_out), lambda i: (i, 0)),
        out_shape=jax.ShapeDtypeStruct((npad, d_out), jnp.float32),
    )


def _scale_kernel(npad, d, br):
    nblk = npad // br

    def body(xw_ref, d0_ref, d1_ref, y_ref):
        deg = d0_ref[...] + d1_ref[...] + 1.0
        dinv = lax.rsqrt(deg)
        y_ref[...] = xw_ref[...] * dinv[:, None]

    return pl.pallas_call(
        body,
        grid=(nblk,),
        in_specs=[
            pl.BlockSpec((br, d), lambda i: (i, 0)),
            pl.BlockSpec((br,), lambda i: (i,)),
            pl.BlockSpec((br,), lambda i: (nblk + i,)),
        ],
        out_specs=pl.BlockSpec((br, d), lambda i: (i, 0)),
        out_shape=jax.ShapeDtypeStruct((npad, d), jnp.float32),
    )


def _final_kernel(n, npad, d, br):
    nblk = npad // br

    def body(acc0_ref, acc1_ref, d0_ref, d1_ref, o_ref):
        deg = d0_ref[...] + d1_ref[...] + 1.0
        dinv = lax.rsqrt(deg)
        o_ref[...] = (acc0_ref[...] + acc1_ref[...]) * dinv[:, None]

    return pl.pallas_call(
        body,
        grid=(nblk,),
        in_specs=[
            pl.BlockSpec((br, d), lambda i: (i, 0)),
            pl.BlockSpec((br, d), lambda i: (nblk + i, 0)),
            pl.BlockSpec((br,), lambda i: (i,)),
            pl.BlockSpec((br,), lambda i: (nblk + i,)),
        ],
        out_specs=pl.BlockSpec((br, d), lambda i: (i, 0)),
        out_shape=jax.ShapeDtypeStruct((n, d), jnp.float32),
    )


def kernel(x, edge_index, num_nodes, W):
    n, d_in = x.shape
    d_out = W.shape[0]
    e = edge_index.shape[1]
    del num_nodes

    npad = -(-n // (NS * 16)) * (NS * 16)
    if npad == n:
        npad += NS * 16

    rem = e % PB
    if rem:
        padn = PB - rem
        pad_iota = jnp.arange(padn, dtype=edge_index.dtype)
        pad_blk = jnp.stack([pad_iota % n, n + pad_iota % (npad - n)])
        edge_index = jnp.concatenate([edge_index, pad_blk], axis=1)
    nfull = (e + (PB - rem if rem else 0)) // PB
    base = nfull // NW
    extra = nfull % NW
    wt = W.T

    degp = _deg_kernel(npad, base, extra)(edge_index)

    xw = _matmul_kernel(npad, d_in, d_out, 512)(x, wt)
    y = _scale_kernel(npad, d_out, 512)(xw, degp, degp)

    accp = _scatter_kernel(npad, base, extra, d_out)(edge_index, y)

    out = _final_kernel(n, npad, d_out, 512)(accp, accp, degp, degp)
    return out


# speedup vs baseline: 1.4779x; 1.0313x over previous
import jax
import jax.numpy as jnp
from jax import lax
from jax.experimental import pallas as pl
from jax.experimental.pallas import tpu as pltpu
from jax.experimental.pallas import tpu_sc as plsc

NC = 2
NS = 16
NW = NC * NS
PB = 128


def _ds(off):
    return pl.ds(pl.multiple_of(off, PB), PB)


def _deg_kernel(npad, base, extra):
    mesh = plsc.VectorSubcoreMesh(core_axis_name="c", subcore_axis_name="s")
    rpt = npad // NS
    nbmax = base + (1 if extra else 0)

    KD = 4
    sbase = base // KD
    btail = base - sbase * KD

    def body(ei_hbm, deg_out, idx_v, ones_v, zero_v, deg_sh, sem):
        c = lax.axis_index("c")
        s = lax.axis_index("s")
        wid = c * NS + s
        base_blk = wid * base
        take_extra = s * NC + c < extra
        for i in range(KD * PB // 16):
            ones_v[pl.ds(i * 16, 16)] = jnp.ones((16,), jnp.float32)
        for i in range(rpt // 16):
            zero_v[pl.ds(i * 16, 16)] = jnp.zeros((16,), jnp.float32)
        pltpu.sync_copy(zero_v, deg_sh.at[pl.ds(s * rpt, rpt)])
        pltpu.async_copy(
            ei_hbm.at[1, pl.ds(pl.multiple_of(base_blk * PB, PB), base * PB)],
            idx_v.at[pl.ds(0, base * PB)], sem)
        if extra:
            @pl.when(take_extra)
            def _():
                pltpu.async_copy(
                    ei_hbm.at[1, _ds((NW * base + s * NC + c) * PB)],
                    idx_v.at[pl.ds(base * PB, PB)], sem)
            pltpu.make_async_copy(
                ei_hbm.at[1, pl.ds(0, base * PB)],
                idx_v.at[pl.ds(0, base * PB)], sem).wait()
            @pl.when(take_extra)
            def _():
                pltpu.make_async_copy(ei_hbm.at[1, _ds(0)],
                                      idx_v.at[pl.ds(0, PB)], sem).wait()
        else:
            pltpu.make_async_copy(
                ei_hbm.at[1, pl.ds(0, base * PB)],
                idx_v.at[pl.ds(0, base * PB)], sem).wait()
        plsc.subcore_barrier()

        def blk(g, carry):
            pltpu.sync_copy(
                ones_v,
                deg_sh.at[idx_v.at[pl.ds(pl.multiple_of(g * (KD * PB),
                                                        KD * PB), KD * PB)]],
                add=True)
            return carry

        lax.fori_loop(0, sbase, blk, 0)
        for t in range(btail):
            pltpu.sync_copy(
                ones_v.at[pl.ds(0, PB)],
                deg_sh.at[idx_v.at[pl.ds((sbase * KD + t) * PB, PB)]],
                add=True)
        if extra:
            @pl.when(take_extra)
            def _():
                pltpu.sync_copy(
                    ones_v.at[pl.ds(0, PB)],
                    deg_sh.at[idx_v.at[pl.ds(base * PB, PB)]], add=True)
        plsc.subcore_barrier()
        pltpu.sync_copy(deg_sh.at[pl.ds(s * rpt, rpt)],
                        deg_out.at[pl.ds(c * npad + s * rpt, rpt)])

    return pl.kernel(
        body,
        out_type=jax.ShapeDtypeStruct((NC * npad,), jnp.float32),
        mesh=mesh,
        scratch_types=[
            pltpu.VMEM((nbmax * PB,), jnp.int32),
            pltpu.VMEM((KD * PB,), jnp.float32),
            pltpu.VMEM((rpt,), jnp.float32),
            pltpu.VMEM_SHARED((npad,), jnp.float32),
            pltpu.SemaphoreType.DMA,
        ],
    )


def _scatter_kernel(npad, base, extra, d):
    mesh = plsc.VectorSubcoreMesh(core_axis_name="c", subcore_axis_name="s")
    rpt = npad // NS
    nbq = base // 4
    tailq = base - 4 * nbq

    def body(ei_hbm, y_hbm, out_hbm,
             idxr0_v, idxr1_v, idxr2_v, idxr3_v,
             idxc0_v, idxc1_v, idxc2_v, idxc3_v, msg_v, acc_sh,
             sem, semi0, semi1, semi2, semi3):
        idxr = (idxr0_v, idxr1_v, idxr2_v, idxr3_v)
        idxc = (idxc0_v, idxc1_v, idxc2_v, idxc3_v)
        semi = (semi0, semi1, semi2, semi3)
        c = lax.axis_index("c")
        s = lax.axis_index("s")
        wid = c * NS + s
        base_blk = wid * base
        use_y = (c == 0) == (s < NS // 2)

        @pl.when(use_y)
        def _():
            pltpu.sync_copy(y_hbm.at[pl.ds(s * rpt, rpt)],
                            acc_sh.at[pl.ds(s * rpt, rpt)])

        @pl.when(jnp.logical_not(use_y))
        def _():
            def zrow(r, carry):
                for i in range(d // 16):
                    msg_v[0, r, pl.ds(i * 16, 16)] = jnp.zeros(
                        (16,), jnp.float32)
                return carry

            lax.fori_loop(0, PB, zrow, 0)
            for r in range(rpt // PB):
                pltpu.sync_copy(
                    msg_v.at[0], acc_sh.at[pl.ds(s * rpt + r * PB, PB)])

        plsc.subcore_barrier()

        def off(j):
            return pl.multiple_of((base_blk + j) * PB, PB)

        def start_idx(slot, j, sm):
            pltpu.async_copy(ei_hbm.at[0, pl.ds(off(j), PB)], idxr[slot], sm)
            pltpu.async_copy(ei_hbm.at[1, pl.ds(off(j), PB)], idxc[slot], sm)

        def wait_idx(slot, sm):
            pltpu.make_async_copy(ei_hbm.at[0, _ds(0)], idxr[slot], sm).wait()
            pltpu.make_async_copy(ei_hbm.at[1, _ds(0)], idxc[slot], sm).wait()

        for b in range(2):
            pltpu.sync_copy(ei_hbm.at[0, pl.ds(off(b), PB)], idxr[b])
            pltpu.sync_copy(ei_hbm.at[1, pl.ds(off(b), PB)], idxc[b])
            pltpu.async_copy(y_hbm.at[idxr[b]], msg_v.at[b], sem)
        for b in range(2, 4):
            if base > b:
                start_idx(b, b, semi[b])

        def quad(q, carry):
            for b4 in range(4):
                j = 4 * q + b4
                ms = b4 % 2
                pltpu.make_async_copy(y_hbm.at[idxr[b4]], msg_v.at[ms],
                                      sem).wait()
                pltpu.sync_copy(msg_v.at[ms], acc_sh.at[idxc[b4]],
                                add=True)
                i2 = (b4 + 2) % 4

                @pl.when(j + 2 < base)
                def _():
                    wait_idx(i2, semi[i2])
                    pltpu.async_copy(y_hbm.at[idxr[i2]], msg_v.at[ms], sem)

                @pl.when(j + 4 < base)
                def _():
                    start_idx(b4, j + 4, semi[b4])
            return carry

        lax.fori_loop(0, nbq, quad, 0)
        for t in range(tailq):
            pltpu.make_async_copy(y_hbm.at[idxr[t]], msg_v.at[t % 2],
                                  sem).wait()
            pltpu.sync_copy(msg_v.at[t % 2], acc_sh.at[idxc[t]], add=True)

        def do_block(blkidx):
            pltpu.sync_copy(ei_hbm.at[0, _ds(blkidx * PB)], idxr[0])
            pltpu.sync_copy(ei_hbm.at[1, _ds(blkidx * PB)], idxc[0])
            pltpu.async_copy(y_hbm.at[idxr[0]], msg_v.at[0], sem).wait()
            pltpu.sync_copy(msg_v.at[0], acc_sh.at[idxc[0]], add=True)

        if extra:
            @pl.when(s * NC + c < extra)
            def _():
                do_block(NW * base + s * NC + c)
        plsc.subcore_barrier()
        pltpu.sync_copy(acc_sh.at[pl.ds(s * rpt, rpt)],
                        out_hbm.at[pl.ds(c * npad + s * rpt, rpt)])

    return pl.kernel(
        body,
        out_type=jax.ShapeDtypeStruct((NC * npad, d), jnp.float32),
        mesh=mesh,
        scratch_types=(
            [pltpu.VMEM((PB,), jnp.int32)] * 8
            + [
                pltpu.VMEM((2, PB, d), jnp.float32),
                pltpu.VMEM_SHARED((npad, d), jnp.float32),
            ]
            + [pltpu.SemaphoreType.DMA] * 5
        ),
    )


def _transform_kernel(npad, d_in, d_out, br):
    nblk = npad // br

    def body(x_ref, wt_ref, d0_ref, d1_ref, y_ref):
        deg = d0_ref[...] + d1_ref[...] + 1.0
        dinv = lax.rsqrt(deg)
        xw = jnp.dot(x_ref[...], wt_ref[...],
                     preferred_element_type=jnp.float32)
        y_ref[...] = xw * dinv[:, None]

    return pl.pallas_call(
        body,
        grid=(nblk,),
        in_specs=[
            pl.BlockSpec((br, d_in), lambda i: (i, 0)),
            pl.BlockSpec((d_in, d_out), lambda i: (0, 0)),
            pl.BlockSpec((br,), lambda i: (i,)),
            pl.BlockSpec((br,), lambda i: (nblk + i,)),
        ],
        out_specs=pl.BlockSpec((br, d_out), lambda i: (i, 0)),
        out_shape=jax.ShapeDtypeStruct((npad, d_out), jnp.float32),
    )


def _final_kernel(n, npad, d, br):
    nblk = npad // br

    def body(acc0_ref, acc1_ref, d0_ref, d1_ref, o_ref):
        deg = d0_ref[...] + d1_ref[...] + 1.0
        dinv = lax.rsqrt(deg)
        o_ref[...] = (acc0_ref[...] + acc1_ref[...]) * dinv[:, None]

    return pl.pallas_call(
        body,
        grid=(nblk,),
        in_specs=[
            pl.BlockSpec((br, d), lambda i: (i, 0)),
            pl.BlockSpec((br, d), lambda i: (nblk + i, 0)),
            pl.BlockSpec((br,), lambda i: (i,)),
            pl.BlockSpec((br,), lambda i: (nblk + i,)),
        ],
        out_specs=pl.BlockSpec((br, d), lambda i: (i, 0)),
        out_shape=jax.ShapeDtypeStruct((n, d), jnp.float32),
    )


def kernel(x, edge_index, num_nodes, W):
    n, d_in = x.shape
    d_out = W.shape[0]
    e = edge_index.shape[1]
    del num_nodes

    npad = -(-n // (NS * 16)) * (NS * 16)
    if npad == n:
        npad += NS * 16

    rem = e % PB
    if rem:
        padn = PB - rem
        pad_iota = jnp.arange(padn, dtype=edge_index.dtype)
        pad_blk = jnp.stack([pad_iota % n, n + pad_iota % (npad - n)])
        edge_index = jnp.concatenate([edge_index, pad_blk], axis=1)
    nfull = (e + (PB - rem if rem else 0)) // PB
    base = nfull // NW
    extra = nfull % NW
    wt = W.T

    degp = _deg_kernel(npad, base, extra)(edge_index)

    y = _transform_kernel(npad, d_in, d_out, 512)(x, wt, degp, degp)

    accp = _scatter_kernel(npad, base, extra, d_out)(edge_index, y)

    out = _final_kernel(n, npad, d_out, 512)(accp, accp, degp, degp)
    return out


# speedup vs baseline: 1.5720x; 1.0637x over previous
import jax
import jax.numpy as jnp
from jax import lax
from jax.experimental import pallas as pl
from jax.experimental.pallas import tpu as pltpu
from jax.experimental.pallas import tpu_sc as plsc

NC = 2
NS = 16
NW = NC * NS
PB = 128


def _ds(off):
    return pl.ds(pl.multiple_of(off, PB), PB)


def _deg_kernel(npad, base, extra):
    mesh = plsc.VectorSubcoreMesh(core_axis_name="c", subcore_axis_name="s")
    rpt = npad // NS
    nbmax = base + (1 if extra else 0)

    KD = 4
    sbase = base // KD
    btail = base - sbase * KD

    def body(ei_hbm, deg_out, idx_v, ones_v, zero_v, deg_sh, sem):
        c = lax.axis_index("c")
        s = lax.axis_index("s")
        wid = c * NS + s
        base_blk = wid * base
        take_extra = s * NC + c < extra
        for i in range(KD * PB // 16):
            ones_v[pl.ds(i * 16, 16)] = jnp.ones((16,), jnp.float32)
        for i in range(rpt // 16):
            zero_v[pl.ds(i * 16, 16)] = jnp.zeros((16,), jnp.float32)
        pltpu.sync_copy(zero_v, deg_sh.at[pl.ds(s * rpt, rpt)])
        pltpu.async_copy(
            ei_hbm.at[1, pl.ds(pl.multiple_of(base_blk * PB, PB), base * PB)],
            idx_v.at[pl.ds(0, base * PB)], sem)
        if extra:
            @pl.when(take_extra)
            def _():
                pltpu.async_copy(
                    ei_hbm.at[1, _ds((NW * base + s * NC + c) * PB)],
                    idx_v.at[pl.ds(base * PB, PB)], sem)
            pltpu.make_async_copy(
                ei_hbm.at[1, pl.ds(0, base * PB)],
                idx_v.at[pl.ds(0, base * PB)], sem).wait()
            @pl.when(take_extra)
            def _():
                pltpu.make_async_copy(ei_hbm.at[1, _ds(0)],
                                      idx_v.at[pl.ds(0, PB)], sem).wait()
        else:
            pltpu.make_async_copy(
                ei_hbm.at[1, pl.ds(0, base * PB)],
                idx_v.at[pl.ds(0, base * PB)], sem).wait()
        plsc.subcore_barrier()

        def blk(g, carry):
            pltpu.sync_copy(
                ones_v,
                deg_sh.at[idx_v.at[pl.ds(pl.multiple_of(g * (KD * PB),
                                                        KD * PB), KD * PB)]],
                add=True)
            return carry

        lax.fori_loop(0, sbase, blk, 0)
        for t in range(btail):
            pltpu.sync_copy(
                ones_v.at[pl.ds(0, PB)],
                deg_sh.at[idx_v.at[pl.ds((sbase * KD + t) * PB, PB)]],
                add=True)
        if extra:
            @pl.when(take_extra)
            def _():
                pltpu.sync_copy(
                    ones_v.at[pl.ds(0, PB)],
                    deg_sh.at[idx_v.at[pl.ds(base * PB, PB)]], add=True)
        plsc.subcore_barrier()
        pltpu.sync_copy(deg_sh.at[pl.ds(s * rpt, rpt)],
                        deg_out.at[pl.ds(c * npad + s * rpt, rpt)])

    return pl.kernel(
        body,
        out_type=jax.ShapeDtypeStruct((NC * npad,), jnp.float32),
        mesh=mesh,
        scratch_types=[
            pltpu.VMEM((nbmax * PB,), jnp.int32),
            pltpu.VMEM((KD * PB,), jnp.float32),
            pltpu.VMEM((rpt,), jnp.float32),
            pltpu.VMEM_SHARED((npad,), jnp.float32),
            pltpu.SemaphoreType.DMA,
        ],
    )


def _scatter_kernel(npad, base, extra, d):
    mesh = plsc.VectorSubcoreMesh(core_axis_name="c", subcore_axis_name="s")
    rpt = npad // NS
    nbq = base // 4
    tailq = base - 4 * nbq

    def body(ei_hbm, y_hbm, out_hbm,
             idxr0_v, idxr1_v, idxr2_v, idxr3_v,
             idxc0_v, idxc1_v, idxc2_v, idxc3_v, msg_v, acc_sh,
             sem, semi0, semi1, semi2, semi3):
        idxr = (idxr0_v, idxr1_v, idxr2_v, idxr3_v)
        idxc = (idxc0_v, idxc1_v, idxc2_v, idxc3_v)
        semi = (semi0, semi1, semi2, semi3)
        c = lax.axis_index("c")
        s = lax.axis_index("s")
        wid = c * NS + s
        base_blk = wid * base
        use_y = (c == 0) == (s < NS // 2)

        @pl.when(use_y)
        def _():
            pltpu.sync_copy(y_hbm.at[pl.ds(s * rpt, rpt)],
                            acc_sh.at[pl.ds(s * rpt, rpt)])

        @pl.when(jnp.logical_not(use_y))
        def _():
            def zrow(r, carry):
                for i in range(d // 16):
                    msg_v[0, r, pl.ds(i * 16, 16)] = jnp.zeros(
                        (16,), jnp.float32)
                return carry

            lax.fori_loop(0, PB, zrow, 0)
            for r in range(rpt // PB):
                pltpu.sync_copy(
                    msg_v.at[0], acc_sh.at[pl.ds(s * rpt + r * PB, PB)])

        plsc.subcore_barrier()

        def off(j):
            return pl.multiple_of((base_blk + j) * PB, PB)

        def start_idx(slot, j, sm):
            pltpu.async_copy(ei_hbm.at[0, pl.ds(off(j), PB)], idxr[slot], sm)
            pltpu.async_copy(ei_hbm.at[1, pl.ds(off(j), PB)], idxc[slot], sm)

        def wait_idx(slot, sm):
            pltpu.make_async_copy(ei_hbm.at[0, _ds(0)], idxr[slot], sm).wait()
            pltpu.make_async_copy(ei_hbm.at[1, _ds(0)], idxc[slot], sm).wait()

        for b in range(2):
            pltpu.sync_copy(ei_hbm.at[0, pl.ds(off(b), PB)], idxr[b])
            pltpu.sync_copy(ei_hbm.at[1, pl.ds(off(b), PB)], idxc[b])
            pltpu.async_copy(y_hbm.at[idxr[b]], msg_v.at[b], sem)
        for b in range(2, 4):
            if base > b:
                start_idx(b, b, semi[b])

        def quad(q, carry):
            for b4 in range(4):
                j = 4 * q + b4
                ms = b4 % 2
                pltpu.make_async_copy(y_hbm.at[idxr[b4]], msg_v.at[ms],
                                      sem).wait()
                pltpu.sync_copy(msg_v.at[ms], acc_sh.at[idxc[b4]],
                                add=True)
                i2 = (b4 + 2) % 4

                @pl.when(j + 2 < base)
                def _():
                    wait_idx(i2, semi[i2])
                    pltpu.async_copy(y_hbm.at[idxr[i2]], msg_v.at[ms], sem)

                @pl.when(j + 4 < base)
                def _():
                    start_idx(b4, j + 4, semi[b4])
            return carry

        lax.fori_loop(0, nbq, quad, 0)
        for t in range(tailq):
            pltpu.make_async_copy(y_hbm.at[idxr[t]], msg_v.at[t % 2],
                                  sem).wait()
            pltpu.sync_copy(msg_v.at[t % 2], acc_sh.at[idxc[t]], add=True)

        def do_block(blkidx):
            pltpu.sync_copy(ei_hbm.at[0, _ds(blkidx * PB)], idxr[0])
            pltpu.sync_copy(ei_hbm.at[1, _ds(blkidx * PB)], idxc[0])
            pltpu.async_copy(y_hbm.at[idxr[0]], msg_v.at[0], sem).wait()
            pltpu.sync_copy(msg_v.at[0], acc_sh.at[idxc[0]], add=True)

        if extra:
            @pl.when(s * NC + c < extra)
            def _():
                do_block(NW * base + s * NC + c)
        plsc.subcore_barrier()
        pltpu.sync_copy(acc_sh.at[pl.ds(s * rpt, rpt)],
                        out_hbm.at[pl.ds(c * npad + s * rpt, rpt)])

    return pl.kernel(
        body,
        out_type=jax.ShapeDtypeStruct((NC * npad, d), jnp.float32),
        mesh=mesh,
        scratch_types=(
            [pltpu.VMEM((PB,), jnp.int32)] * 8
            + [
                pltpu.VMEM((2, PB, d), jnp.float32),
                pltpu.VMEM_SHARED((npad, d), jnp.float32),
            ]
            + [pltpu.SemaphoreType.DMA] * 5
        ),
    )


def _transform_kernel(npad, d_in, d_out, br):
    nblk = npad // br

    def body(x_ref, wt_ref, d0_ref, d1_ref, y_ref):
        deg = d0_ref[...] + d1_ref[...] + 1.0
        dinv = lax.rsqrt(deg)
        xw = jnp.dot(x_ref[...], wt_ref[...],
                     preferred_element_type=jnp.float32)
        y_ref[...] = xw * dinv[:, None]

    return pl.pallas_call(
        body,
        grid=(nblk,),
        in_specs=[
            pl.BlockSpec((br, d_in), lambda i: (i, 0)),
            pl.BlockSpec((d_in, d_out), lambda i: (0, 0)),
            pl.BlockSpec((br,), lambda i: (i,)),
            pl.BlockSpec((br,), lambda i: (nblk + i,)),
        ],
        out_specs=pl.BlockSpec((br, d_out), lambda i: (i, 0)),
        out_shape=jax.ShapeDtypeStruct((npad, d_out), jnp.float32),
    )


def _final_kernel(n, npad, d, br):
    nblk = npad // br

    def body(acc0_ref, acc1_ref, d0_ref, d1_ref, o_ref):
        deg = d0_ref[...] + d1_ref[...] + 1.0
        dinv = lax.rsqrt(deg)
        o_ref[...] = (acc0_ref[...] + acc1_ref[...]) * dinv[:, None]

    return pl.pallas_call(
        body,
        grid=(nblk,),
        in_specs=[
            pl.BlockSpec((br, d), lambda i: (i, 0)),
            pl.BlockSpec((br, d), lambda i: (nblk + i, 0)),
            pl.BlockSpec((br,), lambda i: (i,)),
            pl.BlockSpec((br,), lambda i: (nblk + i,)),
        ],
        out_specs=pl.BlockSpec((br, d), lambda i: (i, 0)),
        out_shape=jax.ShapeDtypeStruct((n, d), jnp.float32),
    )


def kernel(x, edge_index, num_nodes, W):
    n, d_in = x.shape
    d_out = W.shape[0]
    e = edge_index.shape[1]
    del num_nodes

    npad = -(-n // (NS * 16)) * (NS * 16)
    if npad == n:
        npad += NS * 16

    rem = e % PB
    if rem:
        padn = PB - rem
        pad_iota = jnp.arange(padn, dtype=edge_index.dtype)
        pad_blk = jnp.stack([pad_iota % n, n + pad_iota % (npad - n)])
        edge_index = jnp.concatenate([edge_index, pad_blk], axis=1)
    nfull = (e + (PB - rem if rem else 0)) // PB
    base = nfull // NW
    extra = nfull % NW
    wt = W.T

    degp = _deg_kernel(npad, base, extra)(edge_index)

    y = _transform_kernel(npad, d_in, d_out, 1024)(x, wt, degp, degp)

    accp = _scatter_kernel(npad, base, extra, d_out)(edge_index, y)

    out = _final_kernel(n, npad, d_out, 1024)(accp, accp, degp, degp)
    return out


# speedup vs baseline: 1.6220x; 1.0318x over previous
import jax
import jax.numpy as jnp
from jax import lax
from jax.experimental import pallas as pl
from jax.experimental.pallas import tpu as pltpu
from jax.experimental.pallas import tpu_sc as plsc

NC = 2
NS = 16
NW = NC * NS
PB = 128


def _ds(off):
    return pl.ds(pl.multiple_of(off, PB), PB)


def _deg_kernel(npad, base, extra):
    mesh = plsc.VectorSubcoreMesh(core_axis_name="c", subcore_axis_name="s")
    rpt = npad // NS
    nbmax = base + (1 if extra else 0)

    KD = 4
    sbase = base // KD
    btail = base - sbase * KD

    def body(ei_hbm, deg_out, idx_v, ones_v, zero_v, deg_sh, sem):
        c = lax.axis_index("c")
        s = lax.axis_index("s")
        wid = c * NS + s
        base_blk = wid * base
        take_extra = s * NC + c < extra
        for i in range(KD * PB // 16):
            ones_v[pl.ds(i * 16, 16)] = jnp.ones((16,), jnp.float32)
        for i in range(rpt // 16):
            zero_v[pl.ds(i * 16, 16)] = jnp.zeros((16,), jnp.float32)
        pltpu.sync_copy(zero_v, deg_sh.at[pl.ds(s * rpt, rpt)])
        pltpu.async_copy(
            ei_hbm.at[1, pl.ds(pl.multiple_of(base_blk * PB, PB), base * PB)],
            idx_v.at[pl.ds(0, base * PB)], sem)
        if extra:
            @pl.when(take_extra)
            def _():
                pltpu.async_copy(
                    ei_hbm.at[1, _ds((NW * base + s * NC + c) * PB)],
                    idx_v.at[pl.ds(base * PB, PB)], sem)
            pltpu.make_async_copy(
                ei_hbm.at[1, pl.ds(0, base * PB)],
                idx_v.at[pl.ds(0, base * PB)], sem).wait()
            @pl.when(take_extra)
            def _():
                pltpu.make_async_copy(ei_hbm.at[1, _ds(0)],
                                      idx_v.at[pl.ds(0, PB)], sem).wait()
        else:
            pltpu.make_async_copy(
                ei_hbm.at[1, pl.ds(0, base * PB)],
                idx_v.at[pl.ds(0, base * PB)], sem).wait()
        plsc.subcore_barrier()

        def blk(g, carry):
            pltpu.sync_copy(
                ones_v,
                deg_sh.at[idx_v.at[pl.ds(pl.multiple_of(g * (KD * PB),
                                                        KD * PB), KD * PB)]],
                add=True)
            return carry

        lax.fori_loop(0, sbase, blk, 0)
        for t in range(btail):
            pltpu.sync_copy(
                ones_v.at[pl.ds(0, PB)],
                deg_sh.at[idx_v.at[pl.ds((sbase * KD + t) * PB, PB)]],
                add=True)
        if extra:
            @pl.when(take_extra)
            def _():
                pltpu.sync_copy(
                    ones_v.at[pl.ds(0, PB)],
                    deg_sh.at[idx_v.at[pl.ds(base * PB, PB)]], add=True)
        plsc.subcore_barrier()
        pltpu.sync_copy(deg_sh.at[pl.ds(s * rpt, rpt)],
                        deg_out.at[pl.ds(c * npad + s * rpt, rpt)])

    return pl.kernel(
        body,
        out_type=jax.ShapeDtypeStruct((NC * npad,), jnp.float32),
        mesh=mesh,
        scratch_types=[
            pltpu.VMEM((nbmax * PB,), jnp.int32),
            pltpu.VMEM((KD * PB,), jnp.float32),
            pltpu.VMEM((rpt,), jnp.float32),
            pltpu.VMEM_SHARED((npad,), jnp.float32),
            pltpu.SemaphoreType.DMA,
        ],
    )


def _scatter_kernel(npad, base, extra, d):
    mesh = plsc.VectorSubcoreMesh(core_axis_name="c", subcore_axis_name="s")
    rpt = npad // NS
    nbq = base // 4
    tailq = base - 4 * nbq

    def body(ei_hbm, y_hbm, out_hbm,
             idxr0_v, idxr1_v, idxr2_v, idxr3_v,
             idxc0_v, idxc1_v, idxc2_v, idxc3_v, msg_v, acc_sh,
             sem, semi0, semi1, semi2, semi3):
        idxr = (idxr0_v, idxr1_v, idxr2_v, idxr3_v)
        idxc = (idxc0_v, idxc1_v, idxc2_v, idxc3_v)
        semi = (semi0, semi1, semi2, semi3)
        c = lax.axis_index("c")
        s = lax.axis_index("s")
        wid = c * NS + s
        base_blk = wid * base
        use_y = (c == 0) == (s < NS // 2)

        @pl.when(use_y)
        def _():
            pltpu.sync_copy(y_hbm.at[pl.ds(s * rpt, rpt)],
                            acc_sh.at[pl.ds(s * rpt, rpt)])

        @pl.when(jnp.logical_not(use_y))
        def _():
            def zrow(r, carry):
                for i in range(d // 16):
                    msg_v[0, r, pl.ds(i * 16, 16)] = jnp.zeros(
                        (16,), jnp.float32)
                return carry

            lax.fori_loop(0, PB, zrow, 0)
            for r in range(rpt // PB):
                pltpu.sync_copy(
                    msg_v.at[0], acc_sh.at[pl.ds(s * rpt + r * PB, PB)])

        plsc.subcore_barrier()

        def off(j):
            return pl.multiple_of((base_blk + j) * PB, PB)

        def start_idx(slot, j, sm):
            pltpu.async_copy(ei_hbm.at[0, pl.ds(off(j), PB)], idxr[slot], sm)
            pltpu.async_copy(ei_hbm.at[1, pl.ds(off(j), PB)], idxc[slot], sm)

        def wait_idx(slot, sm):
            pltpu.make_async_copy(ei_hbm.at[0, _ds(0)], idxr[slot], sm).wait()
            pltpu.make_async_copy(ei_hbm.at[1, _ds(0)], idxc[slot], sm).wait()

        for b in range(2):
            pltpu.sync_copy(ei_hbm.at[0, pl.ds(off(b), PB)], idxr[b])
            pltpu.sync_copy(ei_hbm.at[1, pl.ds(off(b), PB)], idxc[b])
            pltpu.async_copy(y_hbm.at[idxr[b]], msg_v.at[b], sem)
        for b in range(2, 4):
            if base > b:
                start_idx(b, b, semi[b])

        def quad(q, carry):
            for b4 in range(4):
                j = 4 * q + b4
                ms = b4 % 2
                pltpu.make_async_copy(y_hbm.at[idxr[b4]], msg_v.at[ms],
                                      sem).wait()
                pltpu.sync_copy(msg_v.at[ms], acc_sh.at[idxc[b4]],
                                add=True)
                i2 = (b4 + 2) % 4

                @pl.when(j + 2 < base)
                def _():
                    wait_idx(i2, semi[i2])
                    pltpu.async_copy(y_hbm.at[idxr[i2]], msg_v.at[ms], sem)

                @pl.when(j + 4 < base)
                def _():
                    start_idx(b4, j + 4, semi[b4])
            return carry

        lax.fori_loop(0, nbq, quad, 0)
        for t in range(tailq):
            pltpu.make_async_copy(y_hbm.at[idxr[t]], msg_v.at[t % 2],
                                  sem).wait()
            pltpu.sync_copy(msg_v.at[t % 2], acc_sh.at[idxc[t]], add=True)

        def do_block(blkidx):
            pltpu.sync_copy(ei_hbm.at[0, _ds(blkidx * PB)], idxr[0])
            pltpu.sync_copy(ei_hbm.at[1, _ds(blkidx * PB)], idxc[0])
            pltpu.async_copy(y_hbm.at[idxr[0]], msg_v.at[0], sem).wait()
            pltpu.sync_copy(msg_v.at[0], acc_sh.at[idxc[0]], add=True)

        if extra:
            @pl.when(s * NC + c < extra)
            def _():
                do_block(NW * base + s * NC + c)
        plsc.subcore_barrier()
        pltpu.sync_copy(acc_sh.at[pl.ds(s * rpt, rpt)],
                        out_hbm.at[pl.ds(c * npad + s * rpt, rpt)])

    return pl.kernel(
        body,
        out_type=jax.ShapeDtypeStruct((NC * npad, d), jnp.float32),
        mesh=mesh,
        scratch_types=(
            [pltpu.VMEM((PB,), jnp.int32)] * 8
            + [
                pltpu.VMEM((2, PB, d), jnp.float32),
                pltpu.VMEM_SHARED((npad, d), jnp.float32),
            ]
            + [pltpu.SemaphoreType.DMA] * 5
        ),
    )


def _transform_kernel(npad, d_in, d_out, br):
    nblk = npad // br

    def body(x_ref, wt_ref, d0_ref, d1_ref, y_ref):
        deg = d0_ref[...] + d1_ref[...] + 1.0
        dinv = lax.rsqrt(deg)
        xw = jnp.dot(x_ref[...], wt_ref[...],
                     preferred_element_type=jnp.float32)
        y_ref[...] = xw * dinv[:, None]

    return pl.pallas_call(
        body,
        grid=(nblk,),
        in_specs=[
            pl.BlockSpec((br, d_in), lambda i: (i, 0)),
            pl.BlockSpec((d_in, d_out), lambda i: (0, 0)),
            pl.BlockSpec((br,), lambda i: (i,)),
            pl.BlockSpec((br,), lambda i: (nblk + i,)),
        ],
        out_specs=pl.BlockSpec((br, d_out), lambda i: (i, 0)),
        out_shape=jax.ShapeDtypeStruct((npad, d_out), jnp.float32),
    )


def _final_kernel(n, npad, d, br):
    nblk = npad // br

    def body(acc0_ref, acc1_ref, d0_ref, d1_ref, o_ref):
        deg = d0_ref[...] + d1_ref[...] + 1.0
        dinv = lax.rsqrt(deg)
        o_ref[...] = (acc0_ref[...] + acc1_ref[...]) * dinv[:, None]

    return pl.pallas_call(
        body,
        grid=(nblk,),
        in_specs=[
            pl.BlockSpec((br, d), lambda i: (i, 0)),
            pl.BlockSpec((br, d), lambda i: (nblk + i, 0)),
            pl.BlockSpec((br,), lambda i: (i,)),
            pl.BlockSpec((br,), lambda i: (nblk + i,)),
        ],
        out_specs=pl.BlockSpec((br, d), lambda i: (i, 0)),
        out_shape=jax.ShapeDtypeStruct((n, d), jnp.float32),
    )


def kernel(x, edge_index, num_nodes, W):
    n, d_in = x.shape
    d_out = W.shape[0]
    e = edge_index.shape[1]
    del num_nodes

    npad = -(-n // (NS * 16)) * (NS * 16)
    if npad == n:
        npad += NS * 16

    rem = e % PB
    if rem:
        padn = PB - rem
        pad_iota = jnp.arange(padn, dtype=edge_index.dtype)
        pad_blk = jnp.stack([pad_iota % n, n + pad_iota % (npad - n)])
        edge_index = jnp.concatenate([edge_index, pad_blk], axis=1)
    nfull = (e + (PB - rem if rem else 0)) // PB
    base = nfull // NW
    extra = nfull % NW
    wt = W.T

    degp = _deg_kernel(npad, base, extra)(edge_index)

    y = _transform_kernel(npad, d_in, d_out, 2048)(x, wt, degp, degp)

    accp = _scatter_kernel(npad, base, extra, d_out)(edge_index, y)

    out = _final_kernel(n, npad, d_out, 2048)(accp, accp, degp, degp)
    return out
